# baseline (speedup 1.0000x reference)
import jax
import jax.numpy as jnp
from jax.experimental import pallas as pl

_B = 4096
_L = 2048
_BB = 256


def _onehot_block(seq_ref, out_ref):
    s = seq_ref[...]
    for i in range(4):
        out_ref[:, i * _L:(i + 1) * _L] = (s == i).astype(jnp.float32)


def kernel(sequences):
    seq = sequences.astype(jnp.int32)
    out2d = pl.pallas_call(
        _onehot_block,
        grid=(_B // _BB,),
        in_specs=[pl.BlockSpec((_BB, _L), lambda i: (i, 0))],
        out_specs=pl.BlockSpec((_BB, 4 * _L), lambda i: (i, 0)),
        out_shape=jax.ShapeDtypeStruct((_B, 4 * _L), jnp.float32),
    )(seq)
    return out2d.reshape(_B, 4, _L)


# speedup vs baseline: 3.8962x; 3.8962x over previous
import jax
import jax.numpy as jnp
from jax.experimental import pallas as pl

_B = 4096
_L = 2048
_BB = 256


def _onehot_block(seq_ref, out_ref):
    s = seq_ref[...]
    for i in range(4):
        out_ref[:, i, :] = (s == i).astype(jnp.float32)


def kernel(sequences):
    seq = sequences.astype(jnp.int32)
    return pl.pallas_call(
        _onehot_block,
        grid=(_B // _BB,),
        in_specs=[pl.BlockSpec((_BB, _L), lambda i: (i, 0))],
        out_specs=pl.BlockSpec((_BB, 4, _L), lambda i: (i, 0, 0)),
        out_shape=jax.ShapeDtypeStruct((_B, 4, _L), jnp.float32),
    )(seq)


# speedup vs baseline: 3.9693x; 1.0187x over previous
import jax
import jax.numpy as jnp
from jax.experimental import pallas as pl

_B = 4096
_L = 2048
_BB = 512


def _onehot_block(seq_ref, out_ref):
    s = seq_ref[...]
    for i in range(4):
        out_ref[:, i, :] = (s == i).astype(jnp.float32)


def kernel(sequences):
    seq = sequences.astype(jnp.int32)
    return pl.pallas_call(
        _onehot_block,
        grid=(_B // _BB,),
        in_specs=[pl.BlockSpec((_BB, _L), lambda i: (i, 0))],
        out_specs=pl.BlockSpec((_BB, 4, _L), lambda i: (i, 0, 0)),
        out_shape=jax.ShapeDtypeStruct((_B, 4, _L), jnp.float32),
    )(seq)
